# baseline (speedup 1.0000x reference)
import functools

import jax
import jax.numpy as jnp
from jax import lax
from jax.experimental import pallas as pl
from jax.experimental.pallas import tpu as pltpu
from jax.experimental.pallas import tpu_sc as plsc

_SC_CORES = 2
_SC_SUBCORES = 16
_NW = _SC_CORES * _SC_SUBCORES

_B = 1024
_CTX = 20
_D = 64
_DP = 128
_HID = 128
_V = 100000

_IDX_CHUNK = 128
_N_CHUNKS = (_B * _CTX) // _IDX_CHUNK
_CHUNKS_PER_W = _N_CHUNKS // _NW

_V_BLK = 2048
_NV = pl.cdiv(_V, _V_BLK)
_V_PAD = _NV * _V_BLK


def _sc_gather(table, idx_rows):
    mesh = plsc.VectorSubcoreMesh(core_axis_name="c", subcore_axis_name="s")

    @functools.partial(
        pl.kernel,
        mesh=mesh,
        out_type=jax.ShapeDtypeStruct((_N_CHUNKS * _IDX_CHUNK, _DP), jnp.float32),
        scratch_types=[
            pltpu.VMEM((_CHUNKS_PER_W, _IDX_CHUNK), jnp.int32),
            pltpu.VMEM((_CHUNKS_PER_W * _IDX_CHUNK, _DP), jnp.float32),
            pltpu.SemaphoreType.DMA,
        ],
    )
    def gather_kernel(table_hbm, idx_hbm, out_hbm, idx_v, rows_v, sem):
        wid = lax.axis_index("s") * _SC_CORES + lax.axis_index("c")
        base_chunk = wid * _CHUNKS_PER_W
        pltpu.sync_copy(idx_hbm.at[wid], idx_v)
        for j in range(_CHUNKS_PER_W):
            pltpu.async_copy(
                table_hbm.at[idx_v.at[j]],
                rows_v.at[pl.ds(j * _IDX_CHUNK, _IDX_CHUNK)],
                sem,
            ).wait()
        pltpu.sync_copy(
            rows_v,
            out_hbm.at[pl.ds(base_chunk * _IDX_CHUNK, _CHUNKS_PER_W * _IDX_CHUNK)],
        )

    return gather_kernel(table, idx_rows)


def _mlp1_body(g_ref, w1_ref, b1_ref, h_ref):
    x = g_ref[0]
    for c in range(1, _CTX):
        x = x + g_ref[c]
    h = lax.dot_general(
        x, w1_ref[...], (((1,), (1,)), ((), ())), preferred_element_type=jnp.float32
    )
    h_ref[...] = jnp.maximum(h + b1_ref[...], 0.0)


def _logsoftmax_body(h_ref, w2_ref, b2_ref, o_ref, m_ref, s_ref):
    p = pl.program_id(0)
    v = pl.program_id(1)
    logits = (
        lax.dot_general(
            h_ref[...],
            w2_ref[...],
            (((1,), (1,)), ((), ())),
            preferred_element_type=jnp.float32,
        )
        + b2_ref[...]
    )

    @pl.when(p == 0)
    def _phase0():
        bmax = jnp.max(logits, axis=1, keepdims=True)
        bsum = jnp.sum(jnp.exp(logits - bmax), axis=1, keepdims=True)
        first = v == 0
        m_old = m_ref[...]
        s_old = s_ref[...]
        new_m = jnp.where(first, bmax, jnp.maximum(m_old, bmax))
        new_s = jnp.where(
            first, bsum, s_old * jnp.exp(m_old - new_m) + bsum * jnp.exp(bmax - new_m)
        )
        m_ref[...] = new_m
        s_ref[...] = new_s

    @pl.when(p == 1)
    def _phase1():
        o_ref[...] = logits - (m_ref[...] + jnp.log(s_ref[...]))


def kernel(inputs, table, W1, b1, W2, b2):
    idx_rows = inputs.astype(jnp.int32).T.reshape(_NW, _CHUNKS_PER_W, _IDX_CHUNK)
    table_p = jnp.pad(table, ((0, 0), (0, _DP - _D)))
    w1p = jnp.pad(W1, ((0, 0), (0, _DP - _D)))
    gathered = _sc_gather(table_p, idx_rows)
    g3 = gathered.reshape(_CTX, _B, _DP)

    h = pl.pallas_call(
        _mlp1_body,
        out_shape=jax.ShapeDtypeStruct((_B, _HID), jnp.float32),
    )(g3, w1p, b1.reshape(1, _HID))

    w2p = jnp.pad(W2, ((0, _V_PAD - _V), (0, 0)))
    b2p = jnp.pad(b2, (0, _V_PAD - _V), constant_values=-1e30).reshape(1, _V_PAD)

    out = pl.pallas_call(
        _logsoftmax_body,
        grid=(2, _NV),
        in_specs=[
            pl.BlockSpec((_B, _HID), lambda p, v: (0, 0)),
            pl.BlockSpec((_V_BLK, _HID), lambda p, v: (v, 0)),
            pl.BlockSpec((1, _V_BLK), lambda p, v: (0, v)),
        ],
        out_specs=pl.BlockSpec((_B, _V_BLK), lambda p, v: (0, v * p)),
        out_shape=jax.ShapeDtypeStruct((_B, _V), jnp.float32),
        scratch_shapes=[
            pltpu.VMEM((_B, 1), jnp.float32),
            pltpu.VMEM((_B, 1), jnp.float32),
        ],
    )(h, w2p, b2p)
    return out
